# Initial kernel scaffold; baseline (speedup 1.0000x reference)
import jax
import jax.numpy as jnp
from jax.experimental import pallas as pl


def kernel(x, enc_w1, enc_b1, enc_w2, enc_b2, codebooks, dec_w1, dec_b1, dec_w2, dec_b2):
    raise NotImplementedError("write your pallas kernel here")


# speedup vs baseline: 1.2780x; 1.2780x over previous
import functools

import jax
import jax.numpy as jnp
from jax import lax
from jax.experimental import pallas as pl
from jax.experimental.pallas import tpu as pltpu
from jax.experimental.pallas import tpu_sc as plsc

B, D_IN, D_H, D_Z, K = 4096, 512, 1024, 256, 8192
NUM_LVLS = 4
BT = 512
NBT = B // BT
KT = 2048
NKT = K // KT

_BF = jnp.bfloat16
_F32 = jnp.float32


def _mm(a, b):
    return lax.dot_general(a.astype(_BF), b.astype(_BF),
                           (((1,), (0,)), ((), ())),
                           preferred_element_type=_F32)


def _mm_nt(a, b):
    return lax.dot_general(a.astype(_BF), b.astype(_BF),
                           (((1,), (1,)), ((), ())),
                           preferred_element_type=_F32)


def _enc_body(x_ref, w1_ref, b1_ref, w2_ref, b2_ref, ze_ref):
    h = jnp.maximum(_mm(x_ref[...], w1_ref[...]) + b1_ref[...], 0.0)
    ze_ref[...] = _mm(h, w2_ref[...]) + b2_ref[...]


def _encoder(x, w1, b1, w2, b2):
    return pl.pallas_call(
        _enc_body,
        grid=(NBT,),
        in_specs=[
            pl.BlockSpec((BT, D_IN), lambda i: (i, 0)),
            pl.BlockSpec((D_IN, D_H), lambda i: (0, 0)),
            pl.BlockSpec((1, D_H), lambda i: (0, 0)),
            pl.BlockSpec((D_H, D_Z), lambda i: (0, 0)),
            pl.BlockSpec((1, D_Z), lambda i: (0, 0)),
        ],
        out_specs=pl.BlockSpec((BT, D_Z), lambda i: (i, 0)),
        out_shape=jax.ShapeDtypeStruct((B, D_Z), _F32),
    )(x, w1.astype(_F32), b1.reshape(1, D_H), w2, b2.reshape(1, D_Z))


def _dist_body(first_level, r_ref, q_ref, zq_in_ref, cb_ref,
               idx_ref, r_out_ref, zq_out_ref, rr_rows_ref,
               minv_ref, mini_ref, rr_ref):
    k = pl.program_id(1)

    @pl.when(k == 0)
    def _init():
        if first_level:
            r0 = r_ref[...]
        else:
            r0 = r_ref[...] - q_ref[...]
            r_out_ref[...] = r0
            if zq_in_ref is None:
                zq_out_ref[...] = q_ref[...]
            else:
                zq_out_ref[...] = zq_in_ref[...] + q_ref[...]
        rr = jnp.sum(r0 * r0, axis=1, keepdims=True)
        rr_ref[...] = rr
        if not first_level:
            rr_rows_ref[0, 0, :] = rr[:, 0]
        minv_ref[...] = jnp.full((BT, 128), jnp.inf, _F32)
        mini_ref[...] = jnp.zeros((BT, 128), jnp.int32)

    r = r_ref[...] if first_level else r_out_ref[...]
    cb = cb_ref[...]
    mm2 = _mm_nt(r, cb)
    cc = jnp.sum(cb * cb, axis=1)
    d = (rr_ref[...] - 2.0 * mm2) + cc[None, :]

    minv = minv_ref[...]
    mini = mini_ref[...]
    iota = lax.broadcasted_iota(jnp.int32, (BT, 128), 1)
    base = k * KT
    for g in range(KT // 128):
        dg = d[:, g * 128:(g + 1) * 128]
        ig = iota + (base + g * 128)
        better = dg < minv
        minv = jnp.where(better, dg, minv)
        mini = jnp.where(better, ig, mini)
    minv_ref[...] = minv
    mini_ref[...] = mini

    @pl.when(k == NKT - 1)
    def _fin():
        m = jnp.min(minv, axis=1, keepdims=True)
        cand = jnp.where(minv == m, mini, jnp.int32(2 ** 30))
        idx_ref[0, 0, :] = jnp.min(cand, axis=1)


def _dist_level(r, q_prev, zq_in, cb):
    first_level = q_prev is None
    body = functools.partial(_dist_body, first_level)

    rq_spec = pl.BlockSpec((BT, D_Z), lambda i, k: (i, 0))
    row_spec = pl.BlockSpec((1, 1, BT), lambda i, k: (i, 0, 0))

    in_specs = [rq_spec]
    args = [r]
    if first_level:
        body2 = lambda r_ref, cb_ref, *rest: body(r_ref, None, None, cb_ref, *rest)
    elif zq_in is None:
        in_specs.append(rq_spec)
        args.append(q_prev)
        body2 = lambda r_ref, q_ref, cb_ref, *rest: body(r_ref, q_ref, None, cb_ref, *rest)
    else:
        in_specs += [rq_spec, rq_spec]
        args += [q_prev, zq_in]
        body2 = lambda r_ref, q_ref, zq_ref, cb_ref, *rest: body(r_ref, q_ref, zq_ref, cb_ref, *rest)
    in_specs.append(pl.BlockSpec((KT, D_Z), lambda i, k: (k, 0)))
    args.append(cb)

    out = pl.pallas_call(
        body2,
        grid=(NBT, NKT),
        in_specs=in_specs,
        out_specs=[row_spec, rq_spec, rq_spec, row_spec],
        out_shape=[
            jax.ShapeDtypeStruct((NBT, 1, BT), jnp.int32),
            jax.ShapeDtypeStruct((B, D_Z), _F32),
            jax.ShapeDtypeStruct((B, D_Z), _F32),
            jax.ShapeDtypeStruct((NBT, 1, BT), _F32),
        ],
        scratch_shapes=[
            pltpu.VMEM((BT, 128), _F32),
            pltpu.VMEM((BT, 128), jnp.int32),
            pltpu.VMEM((BT, 1), _F32),
        ],
        compiler_params=pltpu.CompilerParams(
            dimension_semantics=("arbitrary", "arbitrary")),
    )(*args)
    idx_rows, r_new, zq_out, vq_rows = out
    return idx_rows.reshape(B), r_new, zq_out, vq_rows


NW = 32
BPW = B // NW


def _sc_gather(table, idx):
    mesh = plsc.VectorSubcoreMesh(core_axis_name="c", subcore_axis_name="s")

    @functools.partial(
        pl.kernel, mesh=mesh,
        out_type=jax.ShapeDtypeStruct((B, D_Z), _F32),
        scratch_types=[
            pltpu.VMEM((BPW,), jnp.int32),
            pltpu.VMEM((BPW, D_Z), _F32),
            pltpu.SemaphoreType.DMA,
        ],
    )
    def k(table_hbm, idx_hbm, out_hbm, idx_v, rows_v, sem):
        wid = lax.axis_index("s") * 2 + lax.axis_index("c")
        base = wid * BPW
        pltpu.sync_copy(idx_hbm.at[pl.ds(base, BPW)], idx_v)
        pltpu.async_copy(table_hbm.at[idx_v], rows_v, sem).wait()
        pltpu.sync_copy(rows_v, out_hbm.at[pl.ds(base, BPW)])

    return k(table, idx)


def _dec_body(x_ref, ze_ref, zq_in_ref, q4_ref, r4_ref,
              w1_ref, b1_ref, w2_ref, b2_ref,
              xhat_ref, rec_rows_ref, vq_rows_ref):
    zq = zq_in_ref[...] + q4_ref[...]
    zq_st = ze_ref[...] + (zq - ze_ref[...])
    rfin = r4_ref[...] - q4_ref[...]
    vq_rows_ref[0, 0, :] = jnp.sum(rfin * rfin, axis=1)
    h2 = jnp.maximum(_mm(zq_st, w1_ref[...]) + b1_ref[...], 0.0)
    xh = _mm(h2, w2_ref[...]) + b2_ref[...]
    xhat_ref[...] = xh
    e = xh - x_ref[...]
    rec_rows_ref[0, 0, :] = jnp.sum(e * e, axis=1)


def _decoder(x, ze, zq_in, q4, r4, w1, b1, w2, b2):
    rq_spec = pl.BlockSpec((BT, D_Z), lambda i: (i, 0))
    row_spec = pl.BlockSpec((1, 1, BT), lambda i: (i, 0, 0))
    return pl.pallas_call(
        _dec_body,
        grid=(NBT,),
        in_specs=[
            pl.BlockSpec((BT, D_IN), lambda i: (i, 0)),
            rq_spec, rq_spec, rq_spec, rq_spec,
            pl.BlockSpec((D_Z, D_H), lambda i: (0, 0)),
            pl.BlockSpec((1, D_H), lambda i: (0, 0)),
            pl.BlockSpec((D_H, D_IN), lambda i: (0, 0)),
            pl.BlockSpec((1, D_IN), lambda i: (0, 0)),
        ],
        out_specs=[
            pl.BlockSpec((BT, D_IN), lambda i: (i, 0)),
            row_spec, row_spec,
        ],
        out_shape=[
            jax.ShapeDtypeStruct((B, D_IN), _F32),
            jax.ShapeDtypeStruct((NBT, 1, BT), _F32),
            jax.ShapeDtypeStruct((NBT, 1, BT), _F32),
        ],
    )(x, ze, zq_in, q4, r4, w1, b1.reshape(1, D_H), w2, b2.reshape(1, D_IN))


def kernel(x, enc_w1, enc_b1, enc_w2, enc_b2, codebooks,
           dec_w1, dec_b1, dec_w2, dec_b2):
    ze = _encoder(x, enc_w1, enc_b1, enc_w2, enc_b2)

    idx1, _, _, _ = _dist_level(ze, None, None, codebooks[0])
    q1 = _sc_gather(codebooks[0], idx1)

    idx2, r2, zq2, vq1 = _dist_level(ze, q1, None, codebooks[1])
    q2 = _sc_gather(codebooks[1], idx2)

    idx3, r3, zq3, vq2 = _dist_level(r2, q2, zq2, codebooks[2])
    q3 = _sc_gather(codebooks[2], idx3)

    idx4, r4, zq4, vq3 = _dist_level(r3, q3, zq3, codebooks[3])
    q4 = _sc_gather(codebooks[3], idx4)

    x_hat, rec_rows, vq4_rows = _decoder(
        x, ze, zq4, q4, r4, dec_w1, dec_b1, dec_w2, dec_b2)

    n = jnp.float32(B * D_Z)
    vq_loss = jnp.float32(0.0)
    for rows in (vq1, vq2, vq3, vq4_rows):
        cl = jnp.sum(rows) / n
        vq_loss = vq_loss + cl + 0.25 * cl
    recon_loss = jnp.sum(rec_rows) / jnp.float32(B * D_IN)
    loss = recon_loss + 0.25 * vq_loss
    codes = jnp.stack([idx1, idx2, idx3, idx4], axis=1)
    return loss, recon_loss, vq_loss, codes, x_hat


# speedup vs baseline: 1.3123x; 1.0268x over previous
import functools

import jax
import jax.numpy as jnp
from jax import lax
from jax.experimental import pallas as pl
from jax.experimental.pallas import tpu as pltpu
from jax.experimental.pallas import tpu_sc as plsc

B, D_IN, D_H, D_Z, K = 4096, 512, 1024, 256, 8192
NUM_LVLS = 4
BT = 512
NBT = B // BT
KT = 2048
NKT = K // KT

_BF = jnp.bfloat16
_F32 = jnp.float32


def _mm(a, b):
    return lax.dot_general(a.astype(_BF), b.astype(_BF),
                           (((1,), (0,)), ((), ())),
                           preferred_element_type=_F32)


def _mm_nt(a, b):
    return lax.dot_general(a.astype(_BF), b.astype(_BF),
                           (((1,), (1,)), ((), ())),
                           preferred_element_type=_F32)


def _enc_body(x_ref, w1_ref, b1_ref, w2_ref, b2_ref, ze_ref):
    h = jnp.maximum(_mm(x_ref[...], w1_ref[...]) + b1_ref[...], 0.0)
    ze_ref[...] = _mm(h, w2_ref[...]) + b2_ref[...]


def _encoder(x, w1, b1, w2, b2):
    return pl.pallas_call(
        _enc_body,
        grid=(NBT,),
        in_specs=[
            pl.BlockSpec((BT, D_IN), lambda i: (i, 0)),
            pl.BlockSpec((D_IN, D_H), lambda i: (0, 0)),
            pl.BlockSpec((1, D_H), lambda i: (0, 0)),
            pl.BlockSpec((D_H, D_Z), lambda i: (0, 0)),
            pl.BlockSpec((1, D_Z), lambda i: (0, 0)),
        ],
        out_specs=pl.BlockSpec((BT, D_Z), lambda i: (i, 0)),
        out_shape=jax.ShapeDtypeStruct((B, D_Z), _F32),
    )(x, w1.astype(_F32), b1.reshape(1, D_H), w2, b2.reshape(1, D_Z))


def _dist_body(first_level, r_ref, q_ref, zq_in_ref, cb_ref,
               idx_ref, r_out_ref, zq_out_ref, rr_rows_ref,
               minv_ref, mini_ref, rr_ref, cbb_ref, cc_ref):
    i = pl.program_id(0)
    k = pl.program_id(1)

    @pl.when(i == 0)
    def _prep():
        cb = cb_ref[...]
        cbb_ref[pl.ds(k * KT, KT), :] = cb.astype(_BF)
        cc_ref[0, pl.ds(k * KT, KT)] = jnp.sum(cb * cb, axis=1)

    @pl.when(k == 0)
    def _init():
        if first_level:
            r0 = r_ref[...]
        else:
            r0 = r_ref[...] - q_ref[...]
            r_out_ref[...] = r0
            if zq_in_ref is None:
                zq_out_ref[...] = q_ref[...]
            else:
                zq_out_ref[...] = zq_in_ref[...] + q_ref[...]
        rr = jnp.sum(r0 * r0, axis=1, keepdims=True)
        rr_ref[...] = rr
        if not first_level:
            rr_rows_ref[0, 0, :] = rr[:, 0]
        minv_ref[...] = jnp.full((BT, 128), jnp.inf, _F32)
        mini_ref[...] = jnp.zeros((BT, 128), jnp.int32)

    r = r_ref[...] if first_level else r_out_ref[...]
    rb = r.astype(_BF)
    cbb = cbb_ref[pl.ds(k * KT, KT), :]
    mm2 = lax.dot_general(rb, cbb, (((1,), (1,)), ((), ())),
                          preferred_element_type=_F32)
    cc = cc_ref[0, pl.ds(k * KT, KT)]
    d = (rr_ref[...] - 2.0 * mm2) + cc[None, :]

    minv = minv_ref[...]
    mini = mini_ref[...]
    for g in range(KT // 128):
        dg = d[:, g * 128:(g + 1) * 128]
        better = dg < minv
        minv = jnp.where(better, dg, minv)
        mini = jnp.where(better, jnp.int32(k * (KT // 128) + g), mini)
    minv_ref[...] = minv
    mini_ref[...] = mini

    @pl.when(k == NKT - 1)
    def _fin():
        m = jnp.min(minv, axis=1, keepdims=True)
        lane = lax.broadcasted_iota(jnp.int32, (BT, 128), 1)
        cand = jnp.where(minv == m, mini * 128 + lane, jnp.int32(2 ** 30))
        idx_ref[0, 0, :] = jnp.min(cand, axis=1)


def _dist_level(r, q_prev, zq_in, cb):
    first_level = q_prev is None
    body = functools.partial(_dist_body, first_level)

    rq_spec = pl.BlockSpec((BT, D_Z), lambda i, k: (i, 0))
    row_spec = pl.BlockSpec((1, 1, BT), lambda i, k: (i, 0, 0))

    in_specs = [rq_spec]
    args = [r]
    if first_level:
        body2 = lambda r_ref, cb_ref, *rest: body(r_ref, None, None, cb_ref, *rest)
    elif zq_in is None:
        in_specs.append(rq_spec)
        args.append(q_prev)
        body2 = lambda r_ref, q_ref, cb_ref, *rest: body(r_ref, q_ref, None, cb_ref, *rest)
    else:
        in_specs += [rq_spec, rq_spec]
        args += [q_prev, zq_in]
        body2 = lambda r_ref, q_ref, zq_ref, cb_ref, *rest: body(r_ref, q_ref, zq_ref, cb_ref, *rest)
    in_specs.append(pl.BlockSpec((KT, D_Z), lambda i, k: (k, 0)))
    args.append(cb)

    out = pl.pallas_call(
        body2,
        grid=(NBT, NKT),
        in_specs=in_specs,
        out_specs=[row_spec, rq_spec, rq_spec, row_spec],
        out_shape=[
            jax.ShapeDtypeStruct((NBT, 1, BT), jnp.int32),
            jax.ShapeDtypeStruct((B, D_Z), _F32),
            jax.ShapeDtypeStruct((B, D_Z), _F32),
            jax.ShapeDtypeStruct((NBT, 1, BT), _F32),
        ],
        scratch_shapes=[
            pltpu.VMEM((BT, 128), _F32),
            pltpu.VMEM((BT, 128), jnp.int32),
            pltpu.VMEM((BT, 1), _F32),
            pltpu.VMEM((K, D_Z), _BF),
            pltpu.VMEM((1, K), _F32),
        ],
        compiler_params=pltpu.CompilerParams(
            dimension_semantics=("arbitrary", "arbitrary")),
    )(*args)
    idx_rows, r_new, zq_out, vq_rows = out
    return idx_rows.reshape(B), r_new, zq_out, vq_rows


NW = 32
BPW = B // NW


def _sc_gather(table, idx):
    mesh = plsc.VectorSubcoreMesh(core_axis_name="c", subcore_axis_name="s")

    @functools.partial(
        pl.kernel, mesh=mesh,
        out_type=jax.ShapeDtypeStruct((B, D_Z), _F32),
        scratch_types=[
            pltpu.VMEM((BPW,), jnp.int32),
            pltpu.VMEM((BPW, D_Z), _F32),
            pltpu.SemaphoreType.DMA,
        ],
    )
    def k(table_hbm, idx_hbm, out_hbm, idx_v, rows_v, sem):
        wid = lax.axis_index("s") * 2 + lax.axis_index("c")
        base = wid * BPW
        pltpu.sync_copy(idx_hbm.at[pl.ds(base, BPW)], idx_v)
        pltpu.async_copy(table_hbm.at[idx_v], rows_v, sem).wait()
        pltpu.sync_copy(rows_v, out_hbm.at[pl.ds(base, BPW)])

    return k(table, idx)


def _dec_body(x_ref, ze_ref, zq_in_ref, q4_ref, r4_ref,
              w1_ref, b1_ref, w2_ref, b2_ref,
              xhat_ref, rec_rows_ref, vq_rows_ref):
    zq = zq_in_ref[...] + q4_ref[...]
    zq_st = ze_ref[...] + (zq - ze_ref[...])
    rfin = r4_ref[...] - q4_ref[...]
    vq_rows_ref[0, 0, :] = jnp.sum(rfin * rfin, axis=1)
    h2 = jnp.maximum(_mm(zq_st, w1_ref[...]) + b1_ref[...], 0.0)
    xh = _mm(h2, w2_ref[...]) + b2_ref[...]
    xhat_ref[...] = xh
    e = xh - x_ref[...]
    rec_rows_ref[0, 0, :] = jnp.sum(e * e, axis=1)


def _decoder(x, ze, zq_in, q4, r4, w1, b1, w2, b2):
    rq_spec = pl.BlockSpec((BT, D_Z), lambda i: (i, 0))
    row_spec = pl.BlockSpec((1, 1, BT), lambda i: (i, 0, 0))
    return pl.pallas_call(
        _dec_body,
        grid=(NBT,),
        in_specs=[
            pl.BlockSpec((BT, D_IN), lambda i: (i, 0)),
            rq_spec, rq_spec, rq_spec, rq_spec,
            pl.BlockSpec((D_Z, D_H), lambda i: (0, 0)),
            pl.BlockSpec((1, D_H), lambda i: (0, 0)),
            pl.BlockSpec((D_H, D_IN), lambda i: (0, 0)),
            pl.BlockSpec((1, D_IN), lambda i: (0, 0)),
        ],
        out_specs=[
            pl.BlockSpec((BT, D_IN), lambda i: (i, 0)),
            row_spec, row_spec,
        ],
        out_shape=[
            jax.ShapeDtypeStruct((B, D_IN), _F32),
            jax.ShapeDtypeStruct((NBT, 1, BT), _F32),
            jax.ShapeDtypeStruct((NBT, 1, BT), _F32),
        ],
    )(x, ze, zq_in, q4, r4, w1, b1.reshape(1, D_H), w2, b2.reshape(1, D_IN))


def kernel(x, enc_w1, enc_b1, enc_w2, enc_b2, codebooks,
           dec_w1, dec_b1, dec_w2, dec_b2):
    ze = _encoder(x, enc_w1, enc_b1, enc_w2, enc_b2)

    idx1, _, _, _ = _dist_level(ze, None, None, codebooks[0])
    q1 = _sc_gather(codebooks[0], idx1)

    idx2, r2, zq2, vq1 = _dist_level(ze, q1, None, codebooks[1])
    q2 = _sc_gather(codebooks[1], idx2)

    idx3, r3, zq3, vq2 = _dist_level(r2, q2, zq2, codebooks[2])
    q3 = _sc_gather(codebooks[2], idx3)

    idx4, r4, zq4, vq3 = _dist_level(r3, q3, zq3, codebooks[3])
    q4 = _sc_gather(codebooks[3], idx4)

    x_hat, rec_rows, vq4_rows = _decoder(
        x, ze, zq4, q4, r4, dec_w1, dec_b1, dec_w2, dec_b2)

    n = jnp.float32(B * D_Z)
    vq_loss = jnp.float32(0.0)
    for rows in (vq1, vq2, vq3, vq4_rows):
        cl = jnp.sum(rows) / n
        vq_loss = vq_loss + cl + 0.25 * cl
    recon_loss = jnp.sum(rec_rows) / jnp.float32(B * D_IN)
    loss = recon_loss + 0.25 * vq_loss
    codes = jnp.stack([idx1, idx2, idx3, idx4], axis=1)
    return loss, recon_loss, vq_loss, codes, x_hat
